# baseline (speedup 1.0000x reference)
import functools

import jax
import jax.numpy as jnp
from jax import lax
from jax.experimental import pallas as pl
from jax.experimental.pallas import tpu as pltpu
from jax.experimental.pallas import tpu_sc as plsc

N, E, CUR, HID, OUT = 10000, 320000, 128, 128, 128
NC, NS, L = 2, 16, 16
NPAD = 10240
ROWS_PT = NPAD // NS
B = 128
NBT = E // B
NB = NBT // NS
XTRA = NBT - NB * NS
UNR = 12
NOUT = NB // UNR

_MESH = plsc.VectorSubcoreMesh(core_axis_name="c", subcore_axis_name="s",
                               num_cores=NC, num_subcores=NS)
_SC_PARAMS = pltpu.CompilerParams(needs_layout_passes=False)


def _edge_split(s):
  e_base = (s * NB + jnp.minimum(s, XTRA)) * B
  return e_base, s < XTRA, e_base + NB * B


def _sc1_body(col_h, ew_h, deg_h, cnt_h,
              cib, ewb, ones_v, nbuf, deg_sh, si0, si1, si2, si3, sp0, sp1):
  c = lax.axis_index("c")
  s = lax.axis_index("s")
  zero16 = jnp.zeros((L,), jnp.float32)
  one16 = jnp.ones((L,), jnp.float32)
  semI = (si0, si1, si2, si3)
  semP = (sp0, sp1)
  e_base, has_x, e_x = _edge_split(s)
  n0 = s * ROWS_PT
  nlast = N - (NS - 1) * ROWS_PT

  def _d_col(e0, slot):
    return pltpu.make_async_copy(col_h.at[pl.ds(e0, B)], cib.at[slot],
                                 semI[slot])

  def _d_ew(e0, slot):
    return pltpu.make_async_copy(ew_h.at[pl.ds(e0, B)],
                                 ewb.at[pl.ds(slot * B, B)], semI[slot])

  def _src(slot, with_ew):
    return ewb.at[pl.ds(slot * B, B)] if with_ew else ones_v

  def _d_deg(slot2, slot4, with_ew):
    return pltpu.make_async_copy(_src(slot4, with_ew),
                                 deg_sh.at[cib.at[slot4]], semP[slot2])

  def _z1(i, _):
    nbuf[pl.ds(i * L, L)] = zero16
    return 0
  lax.fori_loop(0, ROWS_PT // L, _z1, 0)
  pltpu.sync_copy(nbuf, deg_sh.at[pl.ds(n0, ROWS_PT)])
  for q in range(B // L):
    ones_v[pl.ds(q * L, L)] = one16
  plsc.subcore_barrier()

  def _sweep(with_ew):
    for slot, e0 in ((0, e_base), (1, e_base + B)):
      _d_col(e0, slot).start()
      if with_ew:
        _d_ew(e0, slot).start()

    def _p1(o, _):
      for j in range(4):
        k = o * 4 + j
        e0 = e_base + k * B
        @pl.when(k < NB - 2)
        def _():
          _d_col(e0 + 2 * B, (j + 2) % 4).start()
          if with_ew:
            _d_ew(e0 + 2 * B, (j + 2) % 4).start()
        _d_col(e0, j).wait()
        if with_ew:
          _d_ew(e0, j).wait()
        @pl.when(k >= 1)
        def _():
          _d_deg((j + 1) % 2, (j + 3) % 4, with_ew).wait()
        pltpu.async_copy(_src(j, with_ew), deg_sh.at[cib.at[j]],
                         semP[j % 2], add=True)
      return 0
    lax.fori_loop(0, NB // 4, _p1, 0)
    _d_deg(1, 3, with_ew).wait()
    @pl.when(has_x)
    def _():
      pltpu.sync_copy(col_h.at[pl.ds(e_x, B)], cib.at[0])
      if with_ew:
        pltpu.sync_copy(ew_h.at[pl.ds(e_x, B)], ewb.at[pl.ds(0, B)])
      pltpu.sync_copy(_src(0, with_ew), deg_sh.at[cib.at[0]], add=True)

  @pl.when(c == 0)
  def _():
    _sweep(True)

  @pl.when(c == 1)
  def _():
    _sweep(False)
  plsc.subcore_barrier()

  def _wout(dst):
    pltpu.sync_copy(deg_sh.at[pl.ds(n0, ROWS_PT)], nbuf)
    @pl.when(s < NS - 1)
    def _():
      pltpu.sync_copy(nbuf, dst.at[pl.ds(n0, ROWS_PT)])
    @pl.when(s == NS - 1)
    def _():
      pltpu.sync_copy(nbuf.at[pl.ds(0, nlast)], dst.at[pl.ds(n0, nlast)])

  @pl.when(c == 0)
  def _():
    _wout(deg_h)

  @pl.when(c == 1)
  def _():
    _wout(cnt_h)


_sc1_call = pl.kernel(
    _sc1_body,
    out_type=(
        jax.ShapeDtypeStruct((N,), jnp.float32),
        jax.ShapeDtypeStruct((N,), jnp.float32),
    ),
    mesh=_MESH,
    compiler_params=_SC_PARAMS,
    scratch_types=[
        pltpu.VMEM((4, B), jnp.int32),
        pltpu.VMEM((4 * B,), jnp.float32),
        pltpu.VMEM((B,), jnp.float32),
        pltpu.VMEM((ROWS_PT,), jnp.float32),
        pltpu.VMEM_SHARED((NPAD,), jnp.float32),
    ] + [pltpu.SemaphoreType.DMA] * 6,
)


def _sc2_body(pk_h, ew_h, t_h, h_h, aggp_h, aggs_h,
              pkb, ewb, rows, acc_sh,
              si0, si1, si2, si3, sg0, sg1, sg2, ss0, ss1, ss2, se0, se1):
  c = lax.axis_index("c")
  s = lax.axis_index("s")
  zero16 = jnp.zeros((L,), jnp.float32)
  semI = (si0, si1, si2, si3)
  semG = (sg0, sg1, sg2)
  semS = (ss0, ss1, ss2)
  semE = (se0, se1)
  e_base, has_x, e_x = _edge_split(s)
  n0 = s * ROWS_PT
  nlast = N - (NS - 1) * ROWS_PT

  def _d_pk(e0, slot4):
    return pltpu.make_async_copy(pk_h.at[:, pl.ds(e0, B)],
                                 pkb.at[pl.ds(slot4 * 2, 2), :], semI[slot4])

  def _d_ew(e0, slot2):
    return pltpu.make_async_copy(ew_h.at[pl.ds(e0, B)],
                                 ewb.at[pl.ds(slot2 * B, B)], semE[slot2])

  def _d_gath(tab, slot4, slot3):
    return pltpu.make_async_copy(tab.at[pkb.at[slot4 * 2]],
                                 rows.at[pl.ds(slot3 * B, B), :], semG[slot3])

  def _d_scat(slot3, slot4):
    return pltpu.make_async_copy(rows.at[pl.ds(slot3 * B, B), :],
                                 acc_sh.at[pkb.at[slot4 * 2 + 1]], semS[slot3])

  def _z2(i, _):
    for j in range(8):
      rows[i, pl.ds(j * L, L)] = zero16
    return 0
  lax.fori_loop(0, B, _z2, 0)
  zrows = rows.at[pl.ds(0, B), :]
  @pl.when(s < NS - 1)
  def _():
    for b in range(ROWS_PT // B):
      pltpu.sync_copy(zrows, acc_sh.at[pl.ds(n0 + b * B, B), :])
  @pl.when(s == NS - 1)
  def _():
    for b in range(nlast // B):
      pltpu.sync_copy(zrows, acc_sh.at[pl.ds(n0 + b * B, B), :])
    pltpu.sync_copy(rows.at[pl.ds(0, nlast % B), :],
                    acc_sh.at[pl.ds(n0 + (nlast // B) * B, nlast % B), :])
  plsc.subcore_barrier()

  def _mul_block(slot3, slot2):
    def _mul4(ii, _):
      for u in range(4):
        i = 4 * ii + u
        sv = plsc.load_gather(
            ewb, [jnp.full((L,), slot2 * B + i, jnp.int32)])
        for jf in range(8):
          rows[slot3 * B + i, pl.ds(jf * L, L)] = (
              rows[slot3 * B + i, pl.ds(jf * L, L)] * sv)
      return 0
    lax.fori_loop(0, B // 4, _mul4, 0)

  def _main(tab):
    _d_pk(e_base, 0).start()
    _d_pk(e_base + B, 1).start()
    _d_ew(e_base, 0).start()
    _d_pk(e_base, 0).wait()
    _d_gath(tab, 0, 0).start()

    def _mn(o, _):
      for j in range(UNR):
        k = o * UNR + j
        e0 = e_base + k * B
        s3, s4, s2 = j % 3, j % 4, j % 2
        @pl.when(k >= 2)
        def _():
          _d_scat((j + 1) % 3, (j + 2) % 4).wait()
        @pl.when(k < NB - 2)
        def _():
          _d_pk(e0 + 2 * B, (j + 2) % 4).start()
        @pl.when(k < NB - 1)
        def _():
          _d_ew(e0 + B, (j + 1) % 2).start()
          _d_pk(e0 + B, (j + 1) % 4).wait()
          _d_gath(tab, (j + 1) % 4, (j + 1) % 3).start()
        _d_gath(tab, s4, s3).wait()
        _d_ew(e0, s2).wait()
        _mul_block(s3, s2)
        pltpu.async_copy(rows.at[pl.ds(s3 * B, B), :],
                         acc_sh.at[pkb.at[s4 * 2 + 1]], semS[s3], add=True)
      return 0
    lax.fori_loop(0, NOUT, _mn, 0)
    for (s3, s4) in (((UNR - 2) % 3, (UNR - 2) % 4),
                     ((UNR - 1) % 3, (UNR - 1) % 4)):
      _d_scat(s3, s4).wait()
    @pl.when(has_x)
    def _():
      pltpu.sync_copy(pk_h.at[:, pl.ds(e_x, B)], pkb.at[pl.ds(0, 2), :])
      pltpu.sync_copy(ew_h.at[pl.ds(e_x, B)], ewb.at[pl.ds(0, B)])
      pltpu.async_copy(tab.at[pkb.at[0]],
                       rows.at[pl.ds(0, B), :], sg0).wait()
      _mul_block(0, 0)
      pltpu.sync_copy(rows.at[pl.ds(0, B), :], acc_sh.at[pkb.at[1]], add=True)

  @pl.when(c == 0)
  def _():
    _main(t_h)

  @pl.when(c == 1)
  def _():
    _main(h_h)
  plsc.subcore_barrier()

  def _writeout(dst):
    def _chunk(r0, nr, b3):
      pltpu.sync_copy(acc_sh.at[pl.ds(r0, nr), :],
                      rows.at[pl.ds(b3 * B, nr), :])
      pltpu.sync_copy(rows.at[pl.ds(b3 * B, nr), :], dst.at[pl.ds(r0, nr), :])
    @pl.when(s < NS - 1)
    def _():
      for b in range(ROWS_PT // B):
        _chunk(n0 + b * B, B, b % 3)
    @pl.when(s == NS - 1)
    def _():
      for b in range(nlast // B):
        _chunk(n0 + b * B, B, b % 3)
      _chunk(n0 + (nlast // B) * B, nlast % B, (nlast // B) % 3)

  @pl.when(c == 0)
  def _():
    _writeout(aggp_h)

  @pl.when(c == 1)
  def _():
    _writeout(aggs_h)


_sc2_call = pl.kernel(
    _sc2_body,
    out_type=(
        jax.ShapeDtypeStruct((N, OUT), jnp.float32),
        jax.ShapeDtypeStruct((N, HID), jnp.float32),
    ),
    mesh=_MESH,
    compiler_params=_SC_PARAMS,
    scratch_types=[
        pltpu.VMEM((4 * 2, B), jnp.int32),
        pltpu.VMEM((2 * B,), jnp.float32),
        pltpu.VMEM((3 * B, 128), jnp.float32),
        pltpu.VMEM_SHARED((N, 128), jnp.float32),
    ] + [pltpu.SemaphoreType.DMA] * 12,
)


_BN = 2000


def _pre_body(x_ref, wpt_ref, bp_ref, wi_ref, h_ref, t0_ref):
  h = jnp.dot(x_ref[:], wpt_ref[:], preferred_element_type=jnp.float32)
  h = h + bp_ref[:]
  h_ref[:] = h
  t0_ref[:] = jnp.dot(h, wi_ref[:], preferred_element_type=jnp.float32)


@functools.partial(jax.jit)
def _pre_call(x, wpt, bp, wi):
  return pl.pallas_call(
      _pre_body,
      grid=(N // _BN,),
      in_specs=[
          pl.BlockSpec((_BN, CUR), lambda i: (i, 0)),
          pl.BlockSpec((CUR, HID), lambda i: (0, 0)),
          pl.BlockSpec((1, HID), lambda i: (0, 0)),
          pl.BlockSpec((HID, OUT), lambda i: (0, 0)),
      ],
      out_specs=[
          pl.BlockSpec((_BN, HID), lambda i: (i, 0)),
          pl.BlockSpec((_BN, OUT), lambda i: (i, 0)),
      ],
      out_shape=[
          jax.ShapeDtypeStruct((N, HID), jnp.float32),
          jax.ShapeDtypeStruct((N, OUT), jnp.float32),
      ],
  )(x, wpt, bp, wi)


def _mid_body(t0_ref, deg_ref, cnt_ref, t_ref, dinv_ref, icnt_ref):
  deg = deg_ref[:]
  dinv = jnp.where(deg > 0.0, lax.rsqrt(jnp.maximum(deg, 1e-30)), 0.0)
  dinv_ref[:] = dinv
  icnt_ref[:] = 1.0 / jnp.maximum(cnt_ref[:], 1.0)
  t_ref[:] = dinv * t0_ref[:]


@functools.partial(jax.jit)
def _mid_call(t0, deg, cnt):
  return pl.pallas_call(
      _mid_body,
      grid=(N // _BN,),
      in_specs=[
          pl.BlockSpec((_BN, OUT), lambda i: (i, 0)),
          pl.BlockSpec((_BN, 1), lambda i: (i, 0)),
          pl.BlockSpec((_BN, 1), lambda i: (i, 0)),
      ],
      out_specs=[
          pl.BlockSpec((_BN, OUT), lambda i: (i, 0)),
          pl.BlockSpec((_BN, 1), lambda i: (i, 0)),
          pl.BlockSpec((_BN, 1), lambda i: (i, 0)),
      ],
      out_shape=[
          jax.ShapeDtypeStruct((N, OUT), jnp.float32),
          jax.ShapeDtypeStruct((N, 1), jnp.float32),
          jax.ShapeDtypeStruct((N, 1), jnp.float32),
      ],
  )(t0, deg, cnt)


def _post_body(h_ref, aggp_ref, aggs_ref, icnt_ref, dinv_ref, wroot_ref,
               barma_ref, wlt_ref, bl_ref, wrt_ref, out_ref):
  h = h_ref[:]
  arma = dinv_ref[:] * aggp_ref[:] + jnp.dot(
      h, wroot_ref[:], preferred_element_type=jnp.float32) + barma_ref[:]
  arma = jnp.maximum(arma, 0.0)
  mean = aggs_ref[:] * icnt_ref[:]
  sage = (jnp.dot(mean, wlt_ref[:], preferred_element_type=jnp.float32)
          + jnp.dot(h, wrt_ref[:], preferred_element_type=jnp.float32)
          + bl_ref[:])
  h2 = jnp.where(sage > 0.0, sage, jnp.exp(0.01 * sage) - 1.0)
  out_ref[:] = jnp.concatenate([arma, h2], axis=1)


@functools.partial(jax.jit)
def _post_call(h, aggp, aggs, icnt, dinv, wroot, barma, wlt, bl, wrt):
  return pl.pallas_call(
      _post_body,
      grid=(N // _BN,),
      in_specs=[
          pl.BlockSpec((_BN, HID), lambda i: (i, 0)),
          pl.BlockSpec((_BN, OUT), lambda i: (i, 0)),
          pl.BlockSpec((_BN, HID), lambda i: (i, 0)),
          pl.BlockSpec((_BN, 1), lambda i: (i, 0)),
          pl.BlockSpec((_BN, 1), lambda i: (i, 0)),
          pl.BlockSpec((HID, OUT), lambda i: (0, 0)),
          pl.BlockSpec((1, OUT), lambda i: (0, 0)),
          pl.BlockSpec((HID, OUT), lambda i: (0, 0)),
          pl.BlockSpec((1, OUT), lambda i: (0, 0)),
          pl.BlockSpec((HID, OUT), lambda i: (0, 0)),
      ],
      out_specs=pl.BlockSpec((_BN, 2 * OUT), lambda i: (i, 0)),
      out_shape=jax.ShapeDtypeStruct((N, 2 * OUT), jnp.float32),
  )(h, aggp, aggs, icnt, dinv, wroot, barma, wlt, bl, wrt)


def kernel(x, edge_index, edge_weight, Wp, bp, W_init, W_root, b_arma,
           W_l, b_l, W_r):
  deg, cnt = _sc1_call(edge_index[1], edge_weight)
  h, t0 = _pre_call(x, Wp.T, bp[None, :], W_init)
  t, dinv, icnt = _mid_call(t0, deg[:, None], cnt[:, None])
  aggp, aggs = _sc2_call(edge_index, edge_weight, t, h)
  return _post_call(h, aggp, aggs, icnt, dinv,
                    W_root, b_arma[None, :], W_l.T, b_l[None, :], W_r.T)
